# Initial kernel scaffold; baseline (speedup 1.0000x reference)
import jax
import jax.numpy as jnp
from jax.experimental import pallas as pl


def kernel(x, edge_index, edge_attr, edge_index3, edge_attr3, edge_attr4, batch, params):
    raise NotImplementedError("write your pallas kernel here")


# speedup vs baseline: 3.3561x; 3.3561x over previous
import functools

import jax
import jax.numpy as jnp
from jax import lax
from jax.experimental import pallas as pl
from jax.experimental.pallas import tpu as pltpu
from jax.experimental.pallas import tpu_sc as plsc

N = 10000
E = 320000
E3 = 160000
D = 64
CH = 128
NC = 2
NS = 16
NW = NC * NS


def _tc_prologue(x, bn1g, bn1b, W1, b1, bn2g, bn2b, W2, b2, Wm0, bm0):
    def body(x_ref, g1_ref, be1_ref, W1_ref, c1_ref, g2_ref, be2_ref,
             W2_ref, c2_ref, Wm_ref, bm_ref, h_ref, g_ref):
        x = x_ref[...]
        m = jnp.mean(x, axis=0, keepdims=True)
        v = jnp.mean((x - m) * (x - m), axis=0, keepdims=True)
        h = g1_ref[...] * (x - m) / jnp.sqrt(v + 1e-5) + be1_ref[...]
        h = jnp.maximum(jnp.dot(h, W1_ref[...],
                                preferred_element_type=jnp.float32)
                        + c1_ref[...], 0.0)
        m = jnp.mean(h, axis=0, keepdims=True)
        v = jnp.mean((h - m) * (h - m), axis=0, keepdims=True)
        h = g2_ref[...] * (h - m) / jnp.sqrt(v + 1e-5) + be2_ref[...]
        h = jnp.maximum(jnp.dot(h, W2_ref[...],
                                preferred_element_type=jnp.float32)
                        + c2_ref[...], 0.0)
        h_ref[...] = h
        g_ref[...] = jnp.dot(h, Wm_ref[...],
                             preferred_element_type=jnp.float32) + bm_ref[...]

    return pl.pallas_call(
        body,
        out_shape=[jax.ShapeDtypeStruct((N, D), jnp.float32),
                   jax.ShapeDtypeStruct((N, D), jnp.float32)],
    )(x, bn1g.reshape(1, -1), bn1b.reshape(1, -1), W1, b1.reshape(1, -1),
      bn2g.reshape(1, -1), bn2b.reshape(1, -1), W2, b2.reshape(1, -1),
      Wm0, bm0.reshape(1, -1))


def _tc_layer(h, aggp, Wr, br, Wm, bm):
    def body(h_ref, agg_ref, Wr_ref, br_ref, Wm_ref, bm_ref, hn_ref, gn_ref):
        h = h_ref[...]
        agg = agg_ref[0, :, :] + agg_ref[1, :, :]
        hn = jnp.maximum(jnp.dot(h, Wr_ref[...],
                                 preferred_element_type=jnp.float32)
                         + br_ref[...] + agg, 0.0)
        hn_ref[...] = hn
        gn_ref[...] = jnp.dot(hn, Wm_ref[...],
                              preferred_element_type=jnp.float32) + bm_ref[...]

    return pl.pallas_call(
        body,
        out_shape=[jax.ShapeDtypeStruct((N, D), jnp.float32),
                   jax.ShapeDtypeStruct((N, D), jnp.float32)],
    )(h, aggp, Wr, br.reshape(1, -1), Wm, bm.reshape(1, -1))


def _tc_layer_last(h, aggp, Wr, br, Wu, Wv, bh1):
    def body(h_ref, agg_ref, Wr_ref, br_ref, Wu_ref, Wv_ref, bh_ref,
             u_ref, v_ref):
        h = h_ref[...]
        agg = agg_ref[0, :, :] + agg_ref[1, :, :]
        hn = jnp.maximum(jnp.dot(h, Wr_ref[...],
                                 preferred_element_type=jnp.float32)
                         + br_ref[...] + agg, 0.0)
        u_ref[...] = jnp.dot(hn, Wu_ref[...],
                             preferred_element_type=jnp.float32) + bh_ref[...]
        v_ref[...] = jnp.dot(hn, Wv_ref[...],
                             preferred_element_type=jnp.float32)

    return pl.pallas_call(
        body,
        out_shape=[jax.ShapeDtypeStruct((N, D), jnp.float32),
                   jax.ShapeDtypeStruct((N, D), jnp.float32)],
    )(h, aggp, Wr, br.reshape(1, -1), Wu, Wv, bh1.reshape(1, -1))


def _tc_matmul(a, w, bm_rows=4000):
    M, K = a.shape
    C = w.shape[1]
    grid = M // bm_rows

    def body(a_ref, w_ref, o_ref):
        o_ref[...] = jnp.dot(a_ref[...], w_ref[...],
                             preferred_element_type=jnp.float32)

    return pl.pallas_call(
        body,
        grid=(grid,),
        in_specs=[pl.BlockSpec((bm_rows, K), lambda i: (i, 0)),
                  pl.BlockSpec((K, C), lambda i: (0, 0))],
        out_specs=pl.BlockSpec((bm_rows, C), lambda i: (i, 0)),
        out_shape=jax.ShapeDtypeStruct((M, C), jnp.float32),
    )(a, w)


def _tc_final(t, w2, b2, bm_rows=4000):
    M = t.shape[0]
    grid = M // bm_rows

    def body(t_ref, w_ref, b_ref, o_ref):
        o_ref[...] = jnp.dot(t_ref[...], w_ref[...],
                             preferred_element_type=jnp.float32) + b_ref[...]

    return pl.pallas_call(
        body,
        grid=(grid,),
        in_specs=[pl.BlockSpec((bm_rows, D), lambda i: (i, 0)),
                  pl.BlockSpec((D, 1), lambda i: (0, 0)),
                  pl.BlockSpec((1, 1), lambda i: (0, 0))],
        out_specs=pl.BlockSpec((bm_rows, 1), lambda i: (i, 0)),
        out_shape=jax.ShapeDtypeStruct((M, 1), jnp.float32),
    )(t, w2, b2.reshape(1, 1))


def _sc_conv(g, et, src, dst, *, e_total, e_wrap, col_off):
    total_chunks = e_total // CH
    cpw = -(-total_chunks // NW)
    mesh = plsc.VectorSubcoreMesh(core_axis_name="c", subcore_axis_name="s")

    @functools.partial(
        pl.kernel,
        out_type=jax.ShapeDtypeStruct((NC, N, D), jnp.float32),
        mesh=mesh,
        compiler_params=pltpu.CompilerParams(use_tc_tiling_on_sc=False),
        scratch_types=[
            pltpu.VMEM((CH,), jnp.int32),
            pltpu.VMEM((CH,), jnp.int32),
            pltpu.VMEM((CH, D), jnp.float32),
            pltpu.VMEM((CH, D), jnp.float32),
            pltpu.VMEM((CH, D), jnp.float32),
            pltpu.VMEM_SHARED((N, D), jnp.float32),
            pltpu.SemaphoreType.DMA,
        ],
    )
    def k(g_hbm, et_hbm, src_hbm, dst_hbm, out_hbm,
          src_v, dst_v, gbuf, ebuf, mbuf, agg_sh, sem):
        cid = lax.axis_index("c")
        sid = lax.axis_index("s")
        wid = sid * NC + cid
        zero = jnp.zeros((16,), jnp.float32)

        def zrow(r, carry):
            for q in range(4):
                mbuf[r, pl.ds(q * 16, 16)] = zero
            return carry

        lax.fori_loop(0, CH, zrow, 0)
        for kk in range(5):
            base = jnp.minimum(sid * 625 + kk * CH, N - CH)
            pltpu.sync_copy(mbuf, agg_sh.at[pl.ds(base, CH)])
        plsc.subcore_barrier()

        def chunk_body(j, carry):
            c = wid + j * NW

            @pl.when(c < total_chunks)
            def _():
                base = c * CH
                pltpu.sync_copy(src_hbm.at[pl.ds(base, CH)], src_v)
                pltpu.sync_copy(dst_hbm.at[pl.ds(base, CH)], dst_v)
                if e_wrap < e_total:
                    ebase = jnp.where(base < e_wrap, base, base - e_wrap)
                else:
                    ebase = base
                gcp = pltpu.async_copy(g_hbm.at[src_v], gbuf, sem)
                pltpu.sync_copy(
                    et_hbm.at[pl.ds(ebase, CH), pl.ds(col_off, D)], ebuf)
                gcp.wait()

                def row(r, c2):
                    for q in range(4):
                        s = pl.ds(q * 16, 16)
                        mbuf[r, s] = jnp.maximum(gbuf[r, s] + ebuf[r, s], 0.0)
                    return c2

                lax.fori_loop(0, CH, row, 0)
                pltpu.sync_copy(mbuf, agg_sh.at[dst_v], add=True)

            return carry

        lax.fori_loop(0, cpw, chunk_body, 0)
        plsc.subcore_barrier()

        @pl.when(sid == 0)
        def _():
            pltpu.sync_copy(agg_sh, out_hbm.at[cid])

    return k(g, et, src, dst)


def _sc_head(u, v, et, src, dst, *, col_off):
    total_chunks = E3 // CH
    cpw = -(-total_chunks // NW)
    mesh = plsc.VectorSubcoreMesh(core_axis_name="c", subcore_axis_name="s")

    @functools.partial(
        pl.kernel,
        out_type=jax.ShapeDtypeStruct((E3, D), jnp.float32),
        mesh=mesh,
        compiler_params=pltpu.CompilerParams(use_tc_tiling_on_sc=False),
        scratch_types=[
            pltpu.VMEM((CH,), jnp.int32),
            pltpu.VMEM((CH,), jnp.int32),
            pltpu.VMEM((CH, D), jnp.float32),
            pltpu.VMEM((CH, D), jnp.float32),
            pltpu.VMEM((CH, D), jnp.float32),
            pltpu.SemaphoreType.DMA,
            pltpu.SemaphoreType.DMA,
        ],
    )
    def k(u_hbm, v_hbm, et_hbm, src_hbm, dst_hbm, out_hbm,
          src_v, dst_v, ubuf, vbuf, ebuf, sem1, sem2):
        cid = lax.axis_index("c")
        sid = lax.axis_index("s")
        wid = sid * NC + cid

        def chunk_body(j, carry):
            c = wid + j * NW

            @pl.when(c < total_chunks)
            def _():
                base = c * CH
                pltpu.sync_copy(src_hbm.at[pl.ds(base, CH)], src_v)
                pltpu.sync_copy(dst_hbm.at[pl.ds(base, CH)], dst_v)
                cp1 = pltpu.async_copy(u_hbm.at[src_v], ubuf, sem1)
                cp2 = pltpu.async_copy(v_hbm.at[dst_v], vbuf, sem2)
                pltpu.sync_copy(
                    et_hbm.at[pl.ds(base, CH), pl.ds(col_off, D)], ebuf)
                cp1.wait()
                cp2.wait()

                def row(r, c2):
                    for q in range(4):
                        s = pl.ds(q * 16, 16)
                        ubuf[r, s] = jnp.maximum(
                            ubuf[r, s] + vbuf[r, s] + ebuf[r, s], 0.0)
                    return c2

                lax.fori_loop(0, CH, row, 0)
                pltpu.sync_copy(ubuf, out_hbm.at[pl.ds(base, CH)])

            return carry

        lax.fori_loop(0, cpw, chunk_body, 0)

    return k(u, v, et, src, dst)


def kernel(x, edge_index, edge_attr, edge_index3, edge_attr3, edge_attr4,
           batch, params):
    p = params
    src1 = edge_index[0]
    dst1 = edge_index[1]
    s3 = edge_index3[0]
    d3 = edge_index3[1]
    src2 = jnp.concatenate([s3, d3])
    dst2 = jnp.concatenate([d3, s3])

    Wcat1 = jnp.concatenate(
        [p['c1_%d_Wmsg' % i][D:] for i in range(3)], axis=1)
    Wcat2 = jnp.concatenate(
        [p['c2_%d_Wmsg' % i][D:] for i in range(3)]
        + [p['Wh1'][2 * D:]], axis=1)
    Wcat2 = jnp.pad(Wcat2, ((0, 4), (0, 0)))
    temp = jnp.concatenate([edge_attr3, edge_attr4], axis=1)
    temp_p = jnp.pad(temp, ((0, 0), (0, 4)))
    et1 = _tc_matmul(edge_attr, Wcat1)
    et2 = _tc_matmul(temp_p, Wcat2)

    h, g = _tc_prologue(
        x, p['bn1_g'], p['bn1_b'], p['W1'], p['b1'],
        p['bn2_g'], p['bn2_b'], p['W2'], p['b2'],
        p['c1_0_Wmsg'][:D], p['c1_0_bmsg'])

    for i in range(3):
        aggp = _sc_conv(g, et1, src1, dst1,
                        e_total=E, e_wrap=E, col_off=D * i)
        if i < 2:
            h, g = _tc_layer(h, aggp, p['c1_%d_Wroot' % i],
                             p['c1_%d_broot' % i],
                             p['c1_%d_Wmsg' % (i + 1)][:D],
                             p['c1_%d_bmsg' % (i + 1)])
        else:
            h, g = _tc_layer(h, aggp, p['c1_2_Wroot'], p['c1_2_broot'],
                             p['c2_0_Wmsg'][:D], p['c2_0_bmsg'])

    for i in range(3):
        aggp = _sc_conv(g, et2, src2, dst2,
                        e_total=2 * E3, e_wrap=E3, col_off=D * i)
        if i < 2:
            h, g = _tc_layer(h, aggp, p['c2_%d_Wroot' % i],
                             p['c2_%d_broot' % i],
                             p['c2_%d_Wmsg' % (i + 1)][:D],
                             p['c2_%d_bmsg' % (i + 1)])
        else:
            u, v = _tc_layer_last(h, aggp, p['c2_2_Wroot'], p['c2_2_broot'],
                                  p['Wh1'][:D], p['Wh1'][D:2 * D], p['bh1'])

    t = _sc_head(u, v, et2, s3, d3, col_off=3 * D)
    yhat = _tc_final(t, p['Wh2'], p['bh2'])
    return yhat[:, 0]
